# baseline (speedup 1.0000x reference)
import functools

import jax
import jax.numpy as jnp
from jax.experimental import pallas as pl
from jax.experimental.pallas import tpu as pltpu

_MiB = 1024 * 1024


def _se_kernel(x_ref, w1t_ref, b1_ref, w2t_ref, b2_ref, o_ref, *, inv_hw):
    s_blk = x_ref.shape[0]
    for i in range(s_blk):
        x2d = x_ref[i]
        pooled = jnp.sum(x2d, axis=1, keepdims=True) * inv_hw
        h = jnp.dot(w1t_ref[...], pooled,
                    preferred_element_type=jnp.float32) + b1_ref[...]
        h = jnp.maximum(h, 0.0)
        z = jnp.dot(w2t_ref[...], h,
                    preferred_element_type=jnp.float32) + b2_ref[...]
        gate = jax.nn.sigmoid(z)
        o_ref[i] = x2d * gate


def kernel(x, w1, b1, w2, b2):
    B, C, H, W = x.shape
    HW = H * W
    Cr = w1.shape[1]
    dtype = x.dtype

    xr = x.reshape(B, C, HW)
    w1t = w1.T
    w2t = w2.T
    b1c = b1.reshape(Cr, 1)
    b2c = b2.reshape(C, 1)

    s_blk = 1
    grid = (B // s_blk,)
    slab = C * HW * jnp.dtype(dtype).itemsize
    vmem_limit = int(min(0.9 * 64 * _MiB, 4 * s_blk * slab + 4 * _MiB))

    out = pl.pallas_call(
        functools.partial(_se_kernel, inv_hw=1.0 / HW),
        out_shape=jax.ShapeDtypeStruct(xr.shape, dtype),
        grid=grid,
        in_specs=[
            pl.BlockSpec((s_blk, C, HW), lambda b: (b, 0, 0)),
            pl.BlockSpec((Cr, C), lambda b: (0, 0)),
            pl.BlockSpec((Cr, 1), lambda b: (0, 0)),
            pl.BlockSpec((C, Cr), lambda b: (0, 0)),
            pl.BlockSpec((C, 1), lambda b: (0, 0)),
        ],
        out_specs=pl.BlockSpec((s_blk, C, HW), lambda b: (b, 0, 0)),
        compiler_params=pltpu.CompilerParams(
            dimension_semantics=("parallel",),
            vmem_limit_bytes=vmem_limit),
    )(xr, w1t, b1c, w2t, b2c)
    return out.reshape(x.shape)


# speedup vs baseline: 4.7773x; 4.7773x over previous
import functools

import jax
import jax.numpy as jnp
from jax.experimental import pallas as pl
from jax.experimental.pallas import tpu as pltpu

_MiB = 1024 * 1024


def _se_kernel(x_ref, w1_ref, b1_ref, w2_ref, b2_ref, o_ref, *, inv_hw):
    x3 = x_ref[...]
    pooled = jnp.sum(x3, axis=0) * inv_hw
    h = jnp.dot(pooled, w1_ref[...],
                preferred_element_type=jnp.float32) + b1_ref[...]
    h = jnp.maximum(h, 0.0)
    z = jnp.dot(h, w2_ref[...],
                preferred_element_type=jnp.float32) + b2_ref[...]
    gate = jax.nn.sigmoid(z)
    o_ref[...] = x3 * gate[None]


def kernel(x, w1, b1, w2, b2):
    B, C, H, W = x.shape
    HW = H * W
    Cr = w1.shape[1]
    dtype = x.dtype

    xt = x.transpose(2, 3, 0, 1).reshape(HW, B, C)
    b1r = b1.reshape(1, Cr)
    b2r = b2.reshape(1, C)

    bt = 8 if B % 8 == 0 else B
    grid = (B // bt,)
    block_bytes = HW * bt * C * jnp.dtype(dtype).itemsize
    vmem_limit = int(min(0.92 * 64 * _MiB, 4 * block_bytes + 4 * _MiB))

    out = pl.pallas_call(
        functools.partial(_se_kernel, inv_hw=1.0 / HW),
        out_shape=jax.ShapeDtypeStruct((HW, B, C), dtype),
        grid=grid,
        in_specs=[
            pl.BlockSpec((HW, bt, C), lambda g: (0, g, 0)),
            pl.BlockSpec((C, Cr), lambda g: (0, 0)),
            pl.BlockSpec((1, Cr), lambda g: (0, 0)),
            pl.BlockSpec((Cr, C), lambda g: (0, 0)),
            pl.BlockSpec((1, C), lambda g: (0, 0)),
        ],
        out_specs=pl.BlockSpec((HW, bt, C), lambda g: (0, g, 0)),
        compiler_params=pltpu.CompilerParams(
            dimension_semantics=("parallel",),
            vmem_limit_bytes=vmem_limit),
    )(xt, w1, b1r, w2, b2r)

    return out.reshape(H, W, B, C).transpose(2, 3, 0, 1)
